# baseline (speedup 1.0000x reference)
import functools

import jax
import jax.numpy as jnp
from jax import lax
from jax.experimental import pallas as pl
from jax.experimental.pallas import tpu as pltpu
from jax.experimental.pallas import tpu_sc as plsc

_NUM_PARTS = 100
_NUM_K = 9999
_OUT = 64
_B = 4096
_ROWS = _B * _NUM_PARTS

_NC = 2
_NS = 16
_NW = _NC * _NS
_CH = 128
_RPW = _ROWS // _NW
_CPW = _RPW // _CH


def _sc_gather_body(hash_hbm, table_hbm, out_hbm, hash_v, idx_v, rows_v, sem):
    wid = lax.axis_index("s") * _NC + lax.axis_index("c")
    base_chunk = wid * _CPW
    pltpu.sync_copy(hash_hbm.at[wid], hash_v)

    lane = lax.iota(jnp.int32, 16)

    def compute_idx(j, _):
        def one_vec(k, _):
            r0 = j * _CH + k * 16
            p = lax.rem(lane + r0, _NUM_PARTS)
            idx_v[j, pl.ds(k * 16, 16)] = (
                hash_v[j, pl.ds(k * 16, 16)] + p * (_NUM_K + 1)
            )
            return 0
        return lax.fori_loop(0, _CH // 16, one_vec, 0)

    lax.fori_loop(0, _CPW, compute_idx, 0)

    def chunk(j, _):
        pltpu.async_copy(table_hbm.at[idx_v.at[j]], rows_v, sem).wait()
        off = pl.multiple_of((base_chunk + j) * _CH, _CH)
        pltpu.sync_copy(rows_v, out_hbm.at[pl.ds(off, _CH)])
        return 0

    lax.fori_loop(0, _CPW, chunk, 0)


@jax.jit
def _sc_gather(hashes2d, table):
    mesh = plsc.VectorSubcoreMesh(core_axis_name="c", subcore_axis_name="s")
    return pl.kernel(
        _sc_gather_body,
        out_type=jax.ShapeDtypeStruct((_ROWS, _OUT), jnp.float32),
        mesh=mesh,
        scratch_types=[
            pltpu.VMEM((_CPW, _CH), jnp.int32),
            pltpu.VMEM((_CPW, _CH), jnp.int32),
            pltpu.VMEM((_CH, _OUT), jnp.float32),
            pltpu.SemaphoreType.DMA,
        ],
        compiler_params=pltpu.CompilerParams(use_tc_tiling_on_sc=False),
    )(hashes2d, table)


_BM = 1600


def _tc_body(g_ref, pe_ref, w_ref, b_ref, o_ref):
    x = g_ref[...] + pe_ref[...]
    o_ref[...] = lax.dot_general(
        x, w_ref[...], (((1,), (1,)), ((), ())),
        preferred_element_type=jnp.float32,
    ) + b_ref[...]


@jax.jit
def _tc_project(gathered, pe_tiled, W, b2d):
    return pl.pallas_call(
        _tc_body,
        grid=(_ROWS // _BM,),
        in_specs=[
            pl.BlockSpec((_BM, _OUT), lambda i: (i, 0)),
            pl.BlockSpec((_BM, _OUT), lambda i: (0, 0)),
            pl.BlockSpec((_OUT, _OUT), lambda i: (0, 0)),
            pl.BlockSpec((1, _OUT), lambda i: (0, 0)),
        ],
        out_specs=pl.BlockSpec((_BM, _OUT), lambda i: (i, 0)),
        out_shape=jax.ShapeDtypeStruct((_ROWS, _OUT), jnp.float32),
    )(gathered, pe_tiled, W, b2d)


def kernel(hashes, table, pe, W, b):
    hashes3d = hashes.reshape(_NW, _CPW, _CH)
    gathered = _sc_gather(hashes3d, table)
    pe_tiled = jnp.tile(pe, (_BM // _NUM_PARTS, 1))
    out = _tc_project(gathered, pe_tiled, W, b.reshape(1, _OUT))
    return out.reshape(_B, _NUM_PARTS, _OUT)


# speedup vs baseline: 2.3769x; 2.3769x over previous
import jax
import jax.numpy as jnp
from jax import lax
from jax.experimental import pallas as pl
from jax.experimental.pallas import tpu as pltpu
from jax.experimental.pallas import tpu_sc as plsc

_NUM_PARTS = 100
_NUM_K = 9999
_OUT = 64
_B = 4096
_ROWS = _B * _NUM_PARTS
_V = (_NUM_K + 1) * _NUM_PARTS

_NC = 2
_NS = 16
_NW = _NC * _NS
_CH = 128
_RPW = _ROWS // _NW
_CPW = _RPW // _CH


_H = 512000
_BN = 4096


def _proj_body(xa_ref, xb_ref, w_ref, o_ref):
    w = w_ref[...]
    ya = lax.dot_general(xa_ref[...], w, (((0,), (1,)), ((), ())),
                         preferred_element_type=jnp.float32)
    yb = lax.dot_general(xb_ref[...], w, (((0,), (1,)), ((), ())),
                         preferred_element_type=jnp.float32)
    o_ref[:, 0:_OUT] = ya
    o_ref[:, _OUT:2 * _OUT] = yb


@jax.jit
def _tc_project_table(tableT, W):
    nsteps = _H // _BN
    return pl.pallas_call(
        _proj_body,
        grid=(nsteps,),
        in_specs=[
            pl.BlockSpec((_OUT, _BN), lambda j: (0, j)),
            pl.BlockSpec((_OUT, _BN),
                         lambda j, n=nsteps: (0, jnp.minimum(j + n,
                                                             _V // _BN))),
            pl.BlockSpec((_OUT, _OUT), lambda j: (0, 0)),
        ],
        out_specs=pl.BlockSpec((_BN, 2 * _OUT), lambda j: (j, 0)),
        out_shape=jax.ShapeDtypeStruct((_H, 2 * _OUT), jnp.float32),
    )(tableT, tableT, W)


def _sc_gather_body(hash_hbm, table_hbm, out_hbm,
                    hash_v, idx_v, rows0, rows1, sem0, sem1):
    wid = lax.axis_index("s") * _NC + lax.axis_index("c")
    i_base = wid * _RPW
    pltpu.sync_copy(hash_hbm.at[pl.ds(i_base, _RPW)], hash_v)

    def compute_idx(g, _):
        h = hash_v[pl.ds(g * 16, 16)]
        p = (i_base + g * 16) >> 12
        r = h + p * (_NUM_K + 1)
        mask = lax.shift_right_arithmetic(r - _H, 31)
        slot = 2 * r - ((2 * _H - 1) & (mask ^ -1))
        idx_v[g >> 3, pl.ds((g & 7) * 16, 16)] = slot
        return 0

    lax.fori_loop(0, _RPW // 16, compute_idx, 0)

    def _dma(j, rows, sem):
        return pltpu.make_async_copy(table_hbm.at[idx_v.at[j]], rows, sem)

    def _writeout(j, rows):
        off = pl.multiple_of((wid * _CPW + j) * _CH, _CH)
        pltpu.sync_copy(rows, out_hbm.at[pl.ds(off, _CH)])

    _dma(0, rows0, sem0).start()

    def pair(j2, _):
        j = 2 * j2
        _dma(j + 1, rows1, sem1).start()
        _dma(j, rows0, sem0).wait()
        _writeout(j, rows0)

        @pl.when(j2 + 1 < _CPW // 2)
        def _():
            _dma(j + 2, rows0, sem0).start()

        _dma(j + 1, rows1, sem1).wait()
        _writeout(j + 1, rows1)
        return 0

    lax.fori_loop(0, _CPW // 2, pair, 0)


@jax.jit
def _sc_gather(hashes_flat, proj_flat):
    mesh = plsc.VectorSubcoreMesh(core_axis_name="c", subcore_axis_name="s")
    return pl.kernel(
        _sc_gather_body,
        out_type=jax.ShapeDtypeStruct((_ROWS, _OUT), jnp.float32),
        mesh=mesh,
        scratch_types=[
            pltpu.VMEM((_RPW,), jnp.int32),
            pltpu.VMEM((_CPW, _CH), jnp.int32),
            pltpu.VMEM((_CH, _OUT), jnp.float32),
            pltpu.VMEM((_CH, _OUT), jnp.float32),
            pltpu.SemaphoreType.DMA,
            pltpu.SemaphoreType.DMA,
        ],
        compiler_params=pltpu.CompilerParams(use_tc_tiling_on_sc=False),
    )(hashes_flat, proj_flat)


def _out_body(g_ref, pe_ref, w_ref, b_ref, i_ref, o_ref):
    g = g_ref[0]
    eye = i_ref[...]
    pv = lax.dot_general(pe_ref[0], w_ref[...], (((1,), (1,)), ((), ())),
                         preferred_element_type=jnp.float32) + b_ref[...]
    x1 = g[:, 0:_OUT] + pv
    x2 = g[:, _OUT:2 * _OUT] + pv
    t1 = lax.dot_general(eye, x1, (((1,), (1,)), ((), ())),
                         preferred_element_type=jnp.float32)
    t2 = lax.dot_general(eye, x2, (((1,), (1,)), ((), ())),
                         preferred_element_type=jnp.float32)
    o_ref[0, :, 0:_B // 2] = t1
    o_ref[0, :, _B // 2:_B] = t2


@jax.jit
def _tc_output(gathered3, pe3, W, b_col, eye):
    return pl.pallas_call(
        _out_body,
        grid=(_NUM_PARTS,),
        in_specs=[
            pl.BlockSpec((1, _B // 2, 2 * _OUT), lambda p: (p, 0, 0)),
            pl.BlockSpec((1, 1, _OUT), lambda p: (p, 0, 0)),
            pl.BlockSpec((_OUT, _OUT), lambda p: (0, 0)),
            pl.BlockSpec((1, _OUT), lambda p: (0, 0)),
            pl.BlockSpec((_OUT, _OUT), lambda p: (0, 0)),
        ],
        out_specs=pl.BlockSpec((1, _OUT, _B), lambda p: (p, 0, 0)),
        out_shape=jax.ShapeDtypeStruct((_NUM_PARTS, _OUT, _B), jnp.float32),
    )(gathered3, pe3, W, b_col, eye)


def kernel(hashes, table, pe, W, b):
    tableT = table.T
    hashes_pm = (hashes.T.reshape(_NUM_PARTS, 2, _B // 2)
                 .transpose(0, 2, 1).reshape(_ROWS))
    proj2 = _tc_project_table(tableT, W)
    gathered = _sc_gather(hashes_pm, proj2.reshape(2 * _H, _OUT))
    gathered3 = gathered.reshape(_NUM_PARTS, _B // 2, 2 * _OUT)
    pe3 = pe.reshape(_NUM_PARTS, 1, _OUT)
    out_pm = _tc_output(gathered3, pe3, W, b.reshape(1, _OUT),
                        jnp.eye(_OUT, dtype=jnp.float32))
    return jnp.transpose(out_pm, (2, 0, 1))
